# baseline (speedup 1.0000x reference)
import functools

import jax
import jax.numpy as jnp
from jax import lax
from jax.experimental import pallas as pl
from jax.experimental.pallas import tpu as pltpu
from jax.experimental.pallas import tpu_sc as plsc

_BATCH = 16384
_D = 32
_NCLASS = 100000
_NC = 2
_NS = 16
_L = 16
_CH = 1024
_SPAN = _BATCH // _NS

_mesh = plsc.VectorSubcoreMesh(core_axis_name="c", subcore_axis_name="s")


@functools.partial(
    pl.kernel,
    out_type=jax.ShapeDtypeStruct((_NC, _BATCH), jnp.float32),
    mesh=_mesh,
    compiler_params=pltpu.CompilerParams(
        needs_layout_passes=False, use_tc_tiling_on_sc=True,
        disable_bounds_checks=True, disable_semaphore_checks=True,
        skip_device_barrier=True),
    scratch_types=[
        pltpu.VMEM((_NCLASS,), jnp.float32),
        pltpu.VMEM((2, _CH), jnp.int32),
        pltpu.VMEM((2, _CH), jnp.float32),
        pltpu.VMEM((2, _CH), jnp.float32),
        pltpu.VMEM((2, _SPAN), jnp.float32),
        pltpu.VMEM_SHARED((_NS, _BATCH), jnp.float32),
        pltpu.SemaphoreType.DMA,
        pltpu.SemaphoreType.DMA,
        pltpu.SemaphoreType.DMA,
        pltpu.SemaphoreType.DMA,
    ],
)
def _center_loss_sc(xt_hbm, labels_hbm, ct_hbm, out_hbm,
                    crow, labv, xrow, sqv, rbuf, stage,
                    semc, semx, sems0, sems1):
    cid = lax.axis_index("c")
    sid = lax.axis_index("s")
    k = sid * _NC + cid

    ccp = pltpu.async_copy(ct_hbm.at[k], crow, semc)

    nch = _BATCH // _CH

    def fetch(ci):
        c0 = ci * _CH
        b = ci % 2
        lcp = pltpu.async_copy(labels_hbm.at[pl.ds(c0, _CH)], labv.at[b],
                               semx)
        xcp = pltpu.async_copy(xt_hbm.at[k, pl.ds(c0, _CH)], xrow.at[b],
                               semx)
        return lcp, xcp

    _U = 4

    def make_group(b):
        def group(g, carry):
            for u in range(_U):
                s = pl.ds((g * _U + u) * _L, _L)
                lv = labv[b, s]
                cvals = plsc.load_gather(crow, [lv])
                d = xrow[b, s] - cvals
                sqv[b, s] = d * d
            return carry
        return group

    sems = (sems0, sems1)
    pend = fetch(0)
    ccp.wait()
    for ci in range(nch):
        b = ci % 2
        nxt = fetch(ci + 1) if ci + 1 < nch else None
        for cp in pend:
            cp.wait()
        if ci >= 2:
            pltpu.make_async_copy(sqv.at[b], stage.at[sid, pl.ds(0, _CH)],
                                  sems[b]).wait()
        lax.fori_loop(0, _CH // (_L * _U), make_group(b), 0)
        pltpu.async_copy(sqv.at[b], stage.at[sid, pl.ds(ci * _CH, _CH)],
                         sems[b])
        pend = nxt
    for ci in (nch - 2, nch - 1):
        pltpu.make_async_copy(sqv.at[ci % 2],
                              stage.at[sid, pl.ds(0, _CH)],
                              sems[ci % 2]).wait()

    plsc.subcore_barrier()

    b0 = sid * _SPAN
    pltpu.sync_copy(stage.at[0, pl.ds(b0, _SPAN)],
                    sqv.at[0, pl.ds(0, _SPAN)])
    cps = {}
    for r in (1, 2):
        cps[r] = pltpu.async_copy(stage.at[r, pl.ds(b0, _SPAN)],
                                  rbuf.at[(r - 1) % 2], semc)
    for r in range(1, _NS):
        cps[r].wait()
        rb = (r - 1) % 2
        for v in range(_SPAN // _L):
            s = pl.ds(v * _L, _L)
            sqv[0, s] = sqv[0, s] + rbuf[rb, s]
        if r + 2 < _NS:
            cps[r + 2] = pltpu.async_copy(
                stage.at[r + 2, pl.ds(b0, _SPAN)], rbuf.at[rb], semc)
    pltpu.sync_copy(sqv.at[0, pl.ds(0, _SPAN)],
                    out_hbm.at[cid, pl.ds(b0, _SPAN)])


def kernel(x, labels, centers):
    partials = _center_loss_sc(x.T, labels.astype(jnp.int32), centers.T)
    dist = partials[0] + partials[1]
    return jnp.mean(jnp.clip(dist, 1e-12, 1e12))


# speedup vs baseline: 1.0346x; 1.0346x over previous
import functools

import jax
import jax.numpy as jnp
from jax import lax
from jax.experimental import pallas as pl
from jax.experimental.pallas import tpu as pltpu
from jax.experimental.pallas import tpu_sc as plsc

_BATCH = 16384
_D = 32
_NCLASS = 100000
_NC = 2
_NS = 16
_L = 16
_CH = 2048
_SPAN = _BATCH // _NS

_mesh = plsc.VectorSubcoreMesh(core_axis_name="c", subcore_axis_name="s")


@functools.partial(
    pl.kernel,
    out_type=jax.ShapeDtypeStruct((_NC, _BATCH), jnp.float32),
    mesh=_mesh,
    compiler_params=pltpu.CompilerParams(
        needs_layout_passes=False, use_tc_tiling_on_sc=True,
        disable_bounds_checks=True, disable_semaphore_checks=True,
        skip_device_barrier=True),
    scratch_types=[
        pltpu.VMEM((_NCLASS,), jnp.float32),
        pltpu.VMEM((2, _CH), jnp.int32),
        pltpu.VMEM((2, _CH), jnp.float32),
        pltpu.VMEM((2, _CH), jnp.float32),
        pltpu.VMEM((_SPAN,), jnp.float32),
        pltpu.VMEM_SHARED((_NS, _BATCH), jnp.float32),
        pltpu.SemaphoreType.DMA,
        pltpu.SemaphoreType.DMA,
        pltpu.SemaphoreType.DMA,
        pltpu.SemaphoreType.DMA,
    ],
)
def _center_loss_sc(xt_hbm, labels_hbm, ct_hbm, out_hbm,
                    crow, labv, xrow, sqv, rbuf, stage,
                    semc, semx, sems0, sems1):
    cid = lax.axis_index("c")
    sid = lax.axis_index("s")
    k = sid * _NC + cid

    ccp = pltpu.async_copy(ct_hbm.at[k], crow, semc)

    nch = _BATCH // _CH

    def fetch(ci):
        c0 = ci * _CH
        b = ci % 2
        lcp = pltpu.async_copy(labels_hbm.at[pl.ds(c0, _CH)], labv.at[b],
                               semx)
        xcp = pltpu.async_copy(xt_hbm.at[k, pl.ds(c0, _CH)], xrow.at[b],
                               semx)
        return lcp, xcp

    _U = 4

    def make_group(b):
        def group(g, carry):
            for u in range(_U):
                s = pl.ds((g * _U + u) * _L, _L)
                lv = labv[b, s]
                cvals = plsc.load_gather(crow, [lv])
                d = xrow[b, s] - cvals
                sqv[b, s] = d * d
            return carry
        return group

    sems = (sems0, sems1)
    pend = fetch(0)
    ccp.wait()
    for ci in range(nch):
        b = ci % 2
        nxt = fetch(ci + 1) if ci + 1 < nch else None
        for cp in pend:
            cp.wait()
        if ci >= 2:
            pltpu.make_async_copy(sqv.at[b], stage.at[sid, pl.ds(0, _CH)],
                                  sems[b]).wait()
        lax.fori_loop(0, _CH // (_L * _U), make_group(b), 0)
        pltpu.async_copy(sqv.at[b], stage.at[sid, pl.ds(ci * _CH, _CH)],
                         sems[b])
        pend = nxt
    for ci in (nch - 2, nch - 1):
        pltpu.make_async_copy(sqv.at[ci % 2],
                              stage.at[sid, pl.ds(0, _CH)],
                              sems[ci % 2]).wait()

    plsc.subcore_barrier()

    b0 = sid * _SPAN
    pltpu.sync_copy(stage.at[0, pl.ds(b0, _SPAN)],
                    sqv.at[0, pl.ds(0, _SPAN)])
    for r in range(1, _NS):
        pltpu.sync_copy(stage.at[r, pl.ds(b0, _SPAN)], rbuf)
        for v in range(_SPAN // _L):
            s = pl.ds(v * _L, _L)
            sqv[0, s] = sqv[0, s] + rbuf[s]
    pltpu.sync_copy(sqv.at[0, pl.ds(0, _SPAN)],
                    out_hbm.at[cid, pl.ds(b0, _SPAN)])


def kernel(x, labels, centers):
    partials = _center_loss_sc(x.T, labels.astype(jnp.int32), centers.T)
    dist = partials[0] + partials[1]
    return jnp.mean(jnp.clip(dist, 1e-12, 1e12))


# speedup vs baseline: 1.1008x; 1.0640x over previous
import functools

import jax
import jax.numpy as jnp
from jax import lax
from jax.experimental import pallas as pl
from jax.experimental.pallas import tpu as pltpu
from jax.experimental.pallas import tpu_sc as plsc

_BATCH = 16384
_D = 32
_NCLASS = 100000
_NC = 2
_NS = 16
_L = 16
_CH = 2048
_SPAN = _BATCH // _NS

_mesh = plsc.VectorSubcoreMesh(core_axis_name="c", subcore_axis_name="s")


@functools.partial(
    pl.kernel,
    out_type=jax.ShapeDtypeStruct((_NC, _BATCH), jnp.float32),
    mesh=_mesh,
    compiler_params=pltpu.CompilerParams(
        needs_layout_passes=False, use_tc_tiling_on_sc=True,
        disable_bounds_checks=True, disable_semaphore_checks=True,
        skip_device_barrier=True),
    scratch_types=[
        pltpu.VMEM((_NCLASS,), jnp.float32),
        pltpu.VMEM((2, _CH), jnp.int32),
        pltpu.VMEM((2, _CH), jnp.float32),
        pltpu.VMEM((2, _CH), jnp.float32),
        pltpu.VMEM((_SPAN,), jnp.float32),
        pltpu.VMEM_SHARED((_NS, _BATCH), jnp.float32),
        pltpu.SemaphoreType.DMA,
        pltpu.SemaphoreType.DMA,
        pltpu.SemaphoreType.DMA,
        pltpu.SemaphoreType.DMA,
    ],
)
def _center_loss_sc(xt_hbm, labels_hbm, ct_hbm, out_hbm,
                    crow, labv, xrow, sqv, rbuf, stage,
                    semc, semx, sems0, sems1):
    cid = lax.axis_index("c")
    sid = lax.axis_index("s")
    k = sid * _NC + cid

    ccp = pltpu.async_copy(ct_hbm.at[k], crow, semc)

    nch = _BATCH // _CH

    def fetch(ci):
        c0 = ci * _CH
        b = ci % 2
        lcp = pltpu.async_copy(labels_hbm.at[pl.ds(c0, _CH)], labv.at[b],
                               semx)
        xcp = pltpu.async_copy(xt_hbm.at[k, pl.ds(c0, _CH)], xrow.at[b],
                               semx)
        return lcp, xcp

    def make_group(b):
        def group(g, carry):
            lv = labv[b, pl.ds(g * _L, _L)]
            cvals = plsc.load_gather(crow, [lv])
            xvals = xrow[b, pl.ds(g * _L, _L)]
            d = xvals - cvals
            sqv[b, pl.ds(g * _L, _L)] = d * d
            return carry
        return group

    sems = (sems0, sems1)
    pend = fetch(0)
    ccp.wait()
    for ci in range(nch):
        b = ci % 2
        nxt = fetch(ci + 1) if ci + 1 < nch else None
        for cp in pend:
            cp.wait()
        if ci >= 2:
            pltpu.make_async_copy(sqv.at[b], stage.at[sid, pl.ds(0, _CH)],
                                  sems[b]).wait()
        lax.fori_loop(0, _CH // _L, make_group(b), 0)
        pltpu.async_copy(sqv.at[b], stage.at[sid, pl.ds(ci * _CH, _CH)],
                         sems[b])
        pend = nxt
    for ci in (nch - 2, nch - 1):
        pltpu.make_async_copy(sqv.at[ci % 2],
                              stage.at[sid, pl.ds(0, _CH)],
                              sems[ci % 2]).wait()

    plsc.subcore_barrier()

    b0 = sid * _SPAN
    pltpu.sync_copy(stage.at[0, pl.ds(b0, _SPAN)],
                    sqv.at[0, pl.ds(0, _SPAN)])
    for r in range(1, _NS):
        pltpu.sync_copy(stage.at[r, pl.ds(b0, _SPAN)], rbuf)
        for v in range(_SPAN // _L):
            s = pl.ds(v * _L, _L)
            sqv[0, s] = sqv[0, s] + rbuf[s]
    pltpu.sync_copy(sqv.at[0, pl.ds(0, _SPAN)],
                    out_hbm.at[cid, pl.ds(b0, _SPAN)])


def kernel(x, labels, centers):
    partials = _center_loss_sc(x.T, labels.astype(jnp.int32), centers.T)
    dist = partials[0] + partials[1]
    return jnp.mean(jnp.clip(dist, 1e-12, 1e12))
